# Initial kernel scaffold; baseline (speedup 1.0000x reference)
import jax
import jax.numpy as jnp
from jax.experimental import pallas as pl


def kernel(x, table):
    raise NotImplementedError("write your pallas kernel here")


# speedup vs baseline: 3.4259x; 3.4259x over previous
import functools

import jax
import jax.numpy as jnp
from jax import lax
from jax.experimental import pallas as pl
from jax.experimental.pallas import tpu as pltpu
from jax.experimental.pallas import tpu_sc as plsc


def _make_sc_copy(n, s, emb):
    info = plsc.get_sparse_core_info()
    nc, ns = info.num_cores, info.num_subcores
    nw = nc * ns
    assert s % nw == 0
    rows_per_w = s // nw
    ch = rows_per_w
    while ch * emb * 4 * 2 > 448 * 1024:
        ch //= 2
    assert rows_per_w % ch == 0
    nch = rows_per_w // ch
    mesh = plsc.VectorSubcoreMesh(core_axis_name="c", subcore_axis_name="s")

    @functools.partial(
        pl.kernel,
        mesh=mesh,
        out_type=jax.ShapeDtypeStruct((n, s, emb), jnp.float32),
        scratch_types=[
            pltpu.VMEM((ch, emb), jnp.float32),
            pltpu.VMEM((ch, emb), jnp.float32),
            pltpu.SemaphoreType.DMA,
            pltpu.SemaphoreType.DMA,
            pltpu.SemaphoreType.DMA,
            pltpu.SemaphoreType.DMA,
        ],
    )
    def sc_copy(table_hbm, out_hbm, buf0, buf1, rsem0, rsem1, wsem0, wsem1):
        wid = lax.axis_index("s") * nc + lax.axis_index("c")
        base = wid * rows_per_w
        bufs = (buf0, buf1)
        rsems = (rsem0, rsem1)
        wsems = (wsem0, wsem1)

        def read_of(c):
            b = c % 2
            return pltpu.make_async_copy(
                table_hbm.at[pl.ds(base + c * ch, ch)], bufs[b], rsems[b]
            )

        def write_of(c, i):
            b = c % 2
            return pltpu.make_async_copy(
                bufs[b], out_hbm.at[i, pl.ds(base + c * ch, ch)], wsems[b]
            )

        read_of(0).start()
        for c in range(nch):
            read_of(c).wait()
            for i in range(n):
                write_of(c, i).start()
            if c + 1 < nch:
                if c >= 1:
                    for i in range(n):
                        write_of(c - 1, i).wait()
                read_of(c + 1).start()
        for c in range(max(nch - 2, 0), nch):
            for i in range(n):
                write_of(c, i).wait()

    return sc_copy


def kernel(x, table):
    n, s = x.shape
    bptt, emb = table.shape
    fn = _make_sc_copy(n, s, emb)
    return fn(table)


# speedup vs baseline: 3.5499x; 1.0362x over previous
import functools

import jax
import jax.numpy as jnp
from jax import lax
from jax.experimental import pallas as pl
from jax.experimental.pallas import tpu as pltpu
from jax.experimental.pallas import tpu_sc as plsc


def _make_sc_copy(n, s, emb):
    info = plsc.get_sparse_core_info()
    nc, ns = info.num_cores, info.num_subcores
    nw = nc * ns
    assert s % nw == 0
    rows_per_w = s // nw
    nbuf = 3
    ch = rows_per_w
    while ch * emb * 4 * nbuf > 448 * 1024:
        ch //= 2
    assert rows_per_w % ch == 0
    nch = rows_per_w // ch
    mesh = plsc.VectorSubcoreMesh(core_axis_name="c", subcore_axis_name="s")

    @functools.partial(
        pl.kernel,
        mesh=mesh,
        out_type=jax.ShapeDtypeStruct((n, s, emb), jnp.float32),
        scratch_types=(
            [pltpu.VMEM((ch, emb), jnp.float32) for _ in range(nbuf)]
            + [pltpu.SemaphoreType.DMA for _ in range(2 * nbuf)]
        ),
    )
    def sc_copy(table_hbm, out_hbm, *refs):
        bufs = refs[:nbuf]
        rsems = refs[nbuf : 2 * nbuf]
        wsems = refs[2 * nbuf :]
        wid = lax.axis_index("s") * nc + lax.axis_index("c")
        base = wid * rows_per_w

        def read_of(c):
            b = c % nbuf
            return pltpu.make_async_copy(
                table_hbm.at[pl.ds(base + c * ch, ch)], bufs[b], rsems[b]
            )

        def write_of(c, i):
            b = c % nbuf
            return pltpu.make_async_copy(
                bufs[b], out_hbm.at[i, pl.ds(base + c * ch, ch)], wsems[b]
            )

        for c in range(min(nbuf, nch)):
            read_of(c).start()
        for c in range(nch):
            read_of(c).wait()
            for i in range(n):
                write_of(c, i).start()
            if c + nbuf < nch:
                for i in range(n):
                    write_of(c, i).wait()
                read_of(c + nbuf).start()
        for c in range(max(nch - nbuf, 0), nch):
            for i in range(n):
                write_of(c, i).wait()

    return sc_copy


def kernel(x, table):
    n, s = x.shape
    bptt, emb = table.shape
    fn = _make_sc_copy(n, s, emb)
    return fn(table)
